# Initial kernel scaffold; baseline (speedup 1.0000x reference)
import jax
import jax.numpy as jnp
from jax.experimental import pallas as pl


def kernel(x, adj, W0, a0, W1, a1, W2, a2, W3, a3, Wout, aout):
    raise NotImplementedError("write your pallas kernel here")


# speedup vs baseline: 1.6789x; 1.6789x over previous
import functools
import math

import jax
import jax.numpy as jnp
from jax.experimental import pallas as pl
from jax.experimental.pallas import tpu as pltpu

_MASK = -9e15
_BI = 256
_BJ = 2048


def _proj_body(x_ref, w_ref, asrc_ref, adst_ref, wh_ref, src_ref, dst_ref):
    wh = jnp.dot(x_ref[...], w_ref[...], preferred_element_type=jnp.float32)
    wh_ref[...] = wh
    src_ref[...] = jnp.dot(wh, asrc_ref[...], preferred_element_type=jnp.float32)
    dst_ref[...] = jnp.dot(wh, adst_ref[...], preferred_element_type=jnp.float32)


def _proj(x_p, wcat, asrc, adst, bp=512):
    n_p, fin = x_p.shape
    ftot = wcat.shape[1]
    grid = (n_p // bp,)
    return pl.pallas_call(
        _proj_body,
        grid=grid,
        in_specs=[
            pl.BlockSpec((bp, fin), lambda i: (i, 0)),
            pl.BlockSpec((fin, ftot), lambda i: (0, 0)),
            pl.BlockSpec((ftot, 8), lambda i: (0, 0)),
            pl.BlockSpec((ftot, 8), lambda i: (0, 0)),
        ],
        out_specs=[
            pl.BlockSpec((bp, ftot), lambda i: (i, 0)),
            pl.BlockSpec((bp, 8), lambda i: (i, 0)),
            pl.BlockSpec((bp, 8), lambda i: (i, 0)),
        ],
        out_shape=[
            jax.ShapeDtypeStruct((n_p, ftot), jnp.float32),
            jax.ShapeDtypeStruct((n_p, 8), jnp.float32),
            jax.ShapeDtypeStruct((n_p, 8), jnp.float32),
        ],
    )(x_p, wcat, asrc, adst)


def _flash_body(src_ref, dstt_ref, wh_ref, adj_ref, out_ref,
                m_ref, l_ref, acc_ref, *, nh, f, bi, bj, n, nj):
    i = pl.program_id(0)
    j = pl.program_id(1)

    @pl.when(j == 0)
    def _init():
        m_ref[...] = jnp.full_like(m_ref, _MASK)
        l_ref[...] = jnp.zeros_like(l_ref)
        acc_ref[...] = jnp.zeros_like(acc_ref)

    adj = adj_ref[...]
    edge = adj > 0.0
    colbase = j * bj
    col = jax.lax.broadcasted_iota(jnp.int32, (1, bj), 1) + colbase
    validc = col < n

    srcb = src_ref[pl.ds(i * bi, bi), :]
    dstb = dstt_ref[:, pl.ds(colbase, bj)]

    for h in range(nh):
        e = srcb[:, h:h + 1] + dstb[h:h + 1, :]
        e = jnp.where(e > 0.0, e, 0.2 * e)
        e = jnp.where(edge, e, _MASK)
        e = jnp.where(validc, e, -jnp.inf)
        m_old = m_ref[:, h:h + 1]
        m_new = jnp.maximum(m_old, jnp.max(e, axis=1, keepdims=True))
        p = jnp.exp(e - m_new)
        alpha = jnp.exp(m_old - m_new)
        m_ref[:, h:h + 1] = m_new
        l_ref[:, h:h + 1] = l_ref[:, h:h + 1] * alpha + jnp.sum(
            p, axis=1, keepdims=True)
        whb = wh_ref[pl.ds(colbase, bj), h * f:(h + 1) * f]
        acc_ref[:, h * f:(h + 1) * f] = (
            acc_ref[:, h * f:(h + 1) * f] * alpha
            + jnp.dot(p, whb, preferred_element_type=jnp.float32))

    @pl.when(j == nj - 1)
    def _finalize():
        for h in range(nh):
            o = acc_ref[:, h * f:(h + 1) * f] / l_ref[:, h:h + 1]
            o = jnp.where(o > 0.0, o, jnp.exp(jnp.minimum(o, 0.0)) - 1.0)
            out_ref[:, h * f:(h + 1) * f] = o


def _flash(src, dstt, wh, adj, nh, f, out_rows, bi=_BI, bj=_BJ):
    n = adj.shape[0]
    n_p = src.shape[0]
    ni, nj = n_p // bi, n_p // bj
    ftot = nh * f
    body = functools.partial(_flash_body, nh=nh, f=f, bi=bi, bj=bj, n=n, nj=nj)
    return pl.pallas_call(
        body,
        grid=(ni, nj),
        in_specs=[
            pl.BlockSpec((n_p, 8), lambda i, j: (0, 0)),
            pl.BlockSpec((8, n_p), lambda i, j: (0, 0)),
            pl.BlockSpec((n_p, ftot), lambda i, j: (0, 0)),
            pl.BlockSpec((bi, bj), lambda i, j: (i, j)),
        ],
        out_specs=pl.BlockSpec((bi, ftot), lambda i, j: (i, 0)),
        out_shape=jax.ShapeDtypeStruct((out_rows, ftot), jnp.float32),
        scratch_shapes=[
            pltpu.VMEM((bi, 8), jnp.float32),
            pltpu.VMEM((bi, 8), jnp.float32),
            pltpu.VMEM((bi, ftot), jnp.float32),
        ],
        compiler_params=pltpu.CompilerParams(
            dimension_semantics=("parallel", "arbitrary")),
    )(src, dstt, wh, adj)


def _block_diag_a(a_list, f):
    nh = len(a_list)
    asrc = jnp.zeros((nh * f, 8), jnp.float32)
    adst = jnp.zeros((nh * f, 8), jnp.float32)
    for h, a in enumerate(a_list):
        asrc = asrc.at[h * f:(h + 1) * f, h].set(a[:f, 0])
        adst = adst.at[h * f:(h + 1) * f, h].set(a[f:, 0])
    return asrc, adst


def kernel(x, adj, W0, a0, W1, a1, W2, a2, W3, a3, Wout, aout):
    n = x.shape[0]
    f = W0.shape[1]
    lcm = (_BI * _BJ) // math.gcd(_BI, _BJ)
    n_p = ((n + lcm - 1) // lcm) * lcm

    wcat = jnp.concatenate([W0, W1, W2, W3], axis=1)
    asrc1, adst1 = _block_diag_a([a0, a1, a2, a3], f)
    x_p = jnp.pad(x, ((0, n_p - n), (0, 0)))
    wh1, src1, dst1 = _proj(x_p, wcat, asrc1, adst1)
    h1 = _flash(src1, dst1.T, wh1, adj, nh=4, f=f, out_rows=n_p)

    asrc2, adst2 = _block_diag_a([aout], Wout.shape[1])
    wh2, src2, dst2 = _proj(h1, Wout, asrc2, adst2)
    return _flash(src2, dst2.T, wh2, adj, nh=1, f=Wout.shape[1], out_rows=n)


# speedup vs baseline: 2.5214x; 1.5018x over previous
import functools

import jax
import jax.numpy as jnp
from jax.experimental import pallas as pl
from jax.experimental.pallas import tpu as pltpu

_MASK = -9e15
_BI = 400
_BJ = 2048
_BP = 512


def _proj_body(x_ref, w_ref, asrc_ref, adst_ref, wh_ref, src_ref, dst_ref):
    wh = jnp.dot(x_ref[...], w_ref[...], preferred_element_type=jnp.float32)
    wh_ref[...] = wh
    src_ref[...] = jnp.dot(wh, asrc_ref[...], preferred_element_type=jnp.float32)
    dst_ref[...] = jnp.dot(wh, adst_ref[...], preferred_element_type=jnp.float32)


def _proj(x_p, wcat, asrc, adst):
    n_p, fin = x_p.shape
    ftot = wcat.shape[1]
    return pl.pallas_call(
        _proj_body,
        grid=(n_p // _BP,),
        in_specs=[
            pl.BlockSpec((_BP, fin), lambda i: (i, 0)),
            pl.BlockSpec((fin, ftot), lambda i: (0, 0)),
            pl.BlockSpec((ftot, 8), lambda i: (0, 0)),
            pl.BlockSpec((ftot, 8), lambda i: (0, 0)),
        ],
        out_specs=[
            pl.BlockSpec((_BP, ftot), lambda i: (i, 0)),
            pl.BlockSpec((_BP, 8), lambda i: (i, 0)),
            pl.BlockSpec((_BP, 8), lambda i: (i, 0)),
        ],
        out_shape=[
            jax.ShapeDtypeStruct((n_p, ftot), jnp.float32),
            jax.ShapeDtypeStruct((n_p, 8), jnp.float32),
            jax.ShapeDtypeStruct((n_p, 8), jnp.float32),
        ],
    )(x_p, wcat, asrc, adst)


def _flash_body(aux_ref, dstt_ref, wh_ref, vonest_ref,
                adj_ref, out_ref, l_ref, acc_ref, swh_ref,
                *, nh, f, bi, bj, nj, n):
    i = pl.program_id(0)
    j = pl.program_id(1)

    @pl.when(j == 0)
    def _init():
        l_ref[...] = jnp.zeros_like(l_ref)
        acc_ref[...] = jnp.zeros_like(acc_ref)
        swh_ref[...] = jnp.zeros_like(swh_ref)

    edge = adj_ref[...] > 0.0
    colbase = j * bj
    rowaux = aux_ref[pl.ds(i * bi, bi), :]
    srcb = rowaux[:, 0:8]
    mhatb = rowaux[:, 8:16]
    dstb = dstt_ref[:, pl.ds(colbase, bj)]
    vones = aux_ref[pl.ds(colbase, bj), 16:24]
    whall = wh_ref[pl.ds(colbase, bj), :]
    swh_ref[...] += jnp.dot(vonest_ref[:, pl.ds(colbase, bj)], whall,
                            preferred_element_type=jnp.float32)

    for h in range(nh):
        t = srcb[:, h:h + 1] + dstb[h:h + 1, :]
        u = jnp.maximum(t, 0.2 * t)
        v = u - mhatb[:, h:h + 1]
        p = jnp.exp(jnp.where(edge, v, _MASK))
        l_ref[:, h:h + 1] += jnp.dot(
            p, vones, preferred_element_type=jnp.float32)[:, :1]
        acc_ref[:, h * f:(h + 1) * f] += jnp.dot(
            p, whall[:, h * f:(h + 1) * f], preferred_element_type=jnp.float32)

    @pl.when(j == nj - 1)
    def _finalize():
        for h in range(nh):
            l = l_ref[:, h:h + 1]
            o = jnp.where(l > 0.0,
                          acc_ref[:, h * f:(h + 1) * f] / jnp.maximum(l, 1e-30),
                          swh_ref[:1, h * f:(h + 1) * f] * (1.0 / n))
            o = jnp.where(o > 0.0, o, jnp.exp(jnp.minimum(o, 0.0)) - 1.0)
            out_ref[:, h * f:(h + 1) * f] = o


def _flash(src, mhat, dstt, wh, vones, adj, nh, f):
    n = adj.shape[0]
    n_pj = vones.shape[0]
    ni, nj = n // _BI, n_pj // _BJ
    ftot = nh * f
    aux = jnp.concatenate(
        [src, mhat, vones, jnp.zeros((n_pj, 104), jnp.float32)], axis=1)
    vonest = vones[:, :8].T
    body = functools.partial(_flash_body, nh=nh, f=f, bi=_BI, bj=_BJ,
                             nj=nj, n=n)
    return pl.pallas_call(
        body,
        grid=(ni, nj),
        in_specs=[
            pl.BlockSpec(aux.shape, lambda i, j: (0, 0)),
            pl.BlockSpec(dstt.shape, lambda i, j: (0, 0)),
            pl.BlockSpec(wh.shape, lambda i, j: (0, 0)),
            pl.BlockSpec(vonest.shape, lambda i, j: (0, 0)),
            pl.BlockSpec((_BI, _BJ), lambda i, j: (i, j)),
        ],
        out_specs=pl.BlockSpec((_BI, ftot), lambda i, j: (i, 0)),
        out_shape=jax.ShapeDtypeStruct((n, ftot), jnp.float32),
        scratch_shapes=[
            pltpu.VMEM((_BI, 8), jnp.float32),
            pltpu.VMEM((_BI, ftot), jnp.float32),
            pltpu.VMEM((8, ftot), jnp.float32),
        ],
        compiler_params=pltpu.CompilerParams(
            dimension_semantics=("parallel", "arbitrary")),
    )(aux, dstt, wh, vonest, adj)


def _block_diag_a(a_list, f):
    nh = len(a_list)
    asrc = jnp.zeros((nh * f, 8), jnp.float32)
    adst = jnp.zeros((nh * f, 8), jnp.float32)
    for h, a in enumerate(a_list):
        asrc = asrc.at[h * f:(h + 1) * f, h].set(a[:f, 0])
        adst = adst.at[h * f:(h + 1) * f, h].set(a[f:, 0])
    return asrc, adst


def _mhat_of(src, dst):
    t = src + jnp.max(dst, axis=0, keepdims=True)
    return jnp.maximum(t, 0.2 * t)


def kernel(x, adj, W0, a0, W1, a1, W2, a2, W3, a3, Wout, aout):
    n = x.shape[0]
    f = W0.shape[1]
    n_p = ((n + _BJ - 1) // _BJ) * _BJ
    vones = jnp.pad(jnp.ones((n, 8), jnp.float32), ((0, n_p - n), (0, 0)))

    wcat = jnp.concatenate([W0, W1, W2, W3], axis=1)
    asrc1, adst1 = _block_diag_a([a0, a1, a2, a3], f)
    x_p = jnp.pad(x, ((0, n_p - n), (0, 0)))
    wh1, src1, dst1 = _proj(x_p, wcat, asrc1, adst1)
    h1 = _flash(src1, _mhat_of(src1, dst1), dst1.T, wh1, vones, adj,
                nh=4, f=f)

    asrc2, adst2 = _block_diag_a([aout], Wout.shape[1])
    h1_p = jnp.pad(h1, ((0, n_p - n), (0, 0)))
    wh2, src2, dst2 = _proj(h1_p, Wout, asrc2, adst2)
    return _flash(src2, _mhat_of(src2, dst2), dst2.T, wh2, vones, adj,
                  nh=1, f=Wout.shape[1])


# speedup vs baseline: 3.4616x; 1.3729x over previous
import functools

import jax
import jax.numpy as jnp
from jax.experimental import pallas as pl
from jax.experimental.pallas import tpu as pltpu

_BI = 400
_BJ = 2048
_BP = 512


def _proj_body(x_ref, w_ref, asrc_ref, adst_ref,
               wh_ref, src_ref, dst_ref, swh_ref):
    i = pl.program_id(0)
    wh = jnp.dot(x_ref[...], w_ref[...], preferred_element_type=jnp.float32)
    wh_ref[...] = wh
    src_ref[...] = jnp.dot(wh, asrc_ref[...], preferred_element_type=jnp.float32)
    dst_ref[...] = jnp.dot(wh, adst_ref[...], preferred_element_type=jnp.float32)

    @pl.when(i == 0)
    def _init():
        swh_ref[...] = jnp.zeros_like(swh_ref)

    swh_ref[0:1, :] += jnp.sum(wh, axis=0, keepdims=True)


def _proj(x_p, wcat, asrc, adst):
    n_p, fin = x_p.shape
    ftot = wcat.shape[1]
    return pl.pallas_call(
        _proj_body,
        grid=(n_p // _BP,),
        in_specs=[
            pl.BlockSpec((_BP, fin), lambda i: (i, 0)),
            pl.BlockSpec((fin, ftot), lambda i: (0, 0)),
            pl.BlockSpec((ftot, 8), lambda i: (0, 0)),
            pl.BlockSpec((ftot, 8), lambda i: (0, 0)),
        ],
        out_specs=[
            pl.BlockSpec((_BP, ftot), lambda i: (i, 0)),
            pl.BlockSpec((_BP, 8), lambda i: (i, 0)),
            pl.BlockSpec((_BP, 8), lambda i: (i, 0)),
            pl.BlockSpec((8, ftot), lambda i: (0, 0)),
        ],
        out_shape=[
            jax.ShapeDtypeStruct((n_p, ftot), jnp.float32),
            jax.ShapeDtypeStruct((n_p, 8), jnp.float32),
            jax.ShapeDtypeStruct((n_p, 8), jnp.float32),
            jax.ShapeDtypeStruct((8, ftot), jnp.float32),
        ],
    )(x_p, wcat, asrc, adst)


def _flash_body(aux_ref, fcol_ref, whx_ref, swh_ref, adj_ref,
                out_ref, acc_ref, *, nh, f, bi, bj, nj, n):
    i = pl.program_id(0)
    j = pl.program_id(1)

    @pl.when(j == 0)
    def _init():
        acc_ref[...] = jnp.zeros_like(acc_ref)

    colbase = j * bj
    col = jax.lax.broadcasted_iota(jnp.int32, (1, bj), 1) + colbase
    adjm = jnp.where(col < n, adj_ref[...], 0.0)

    rowaux = aux_ref[pl.ds(i * bi, bi), :]
    e1 = rowaux[:, 0:8]
    e2 = rowaux[:, 8:16]
    fcol = fcol_ref[:, pl.ds(colbase, bj)]

    for h in range(nh):
        p = jnp.maximum(e1[:, h:h + 1] * fcol[h:h + 1, :],
                        e2[:, h:h + 1] * fcol[h + 8:h + 9, :])
        pm = (p * adjm).astype(jnp.bfloat16)
        whb = whx_ref[pl.ds(colbase, bj), h * 128:(h + 1) * 128]
        acc_ref[:, h * 128:(h + 1) * 128] += jnp.dot(
            pm, whb, preferred_element_type=jnp.float32)

    @pl.when(j == nj - 1)
    def _finalize():
        for h in range(nh):
            l = acc_ref[:, h * 128 + f:h * 128 + f + 1]
            o = jnp.where(l > 0.0,
                          acc_ref[:, h * 128:h * 128 + f] / jnp.maximum(l, 1e-30),
                          swh_ref[:1, h * f:(h + 1) * f] * (1.0 / n))
            o = jnp.where(o > 0.0, o, jnp.exp(jnp.minimum(o, 0.0)) - 1.0)
            out_ref[:, h * f:(h + 1) * f] = o


def _flash(aux, fcol, whx, swh, adj, nh, f):
    n = adj.shape[0]
    n_pj = whx.shape[0]
    ni, nj = n // _BI, n_pj // _BJ
    body = functools.partial(_flash_body, nh=nh, f=f, bi=_BI, bj=_BJ,
                             nj=nj, n=n)
    return pl.pallas_call(
        body,
        grid=(ni, nj),
        in_specs=[
            pl.BlockSpec(aux.shape, lambda i, j: (0, 0)),
            pl.BlockSpec(fcol.shape, lambda i, j: (0, 0)),
            pl.BlockSpec(whx.shape, lambda i, j: (0, 0)),
            pl.BlockSpec(swh.shape, lambda i, j: (0, 0)),
            pl.BlockSpec((_BI, _BJ), lambda i, j: (i, j)),
        ],
        out_specs=pl.BlockSpec((_BI, nh * f), lambda i, j: (i, 0)),
        out_shape=jax.ShapeDtypeStruct((n, nh * f), jnp.float32),
        scratch_shapes=[
            pltpu.VMEM((_BI, nh * 128), jnp.float32),
        ],
        compiler_params=pltpu.CompilerParams(
            dimension_semantics=("parallel", "arbitrary")),
    )(aux, fcol, whx, swh, adj)


def _block_diag_a(a_list, f):
    nh = len(a_list)
    asrc = jnp.zeros((nh * f, 8), jnp.float32)
    adst = jnp.zeros((nh * f, 8), jnp.float32)
    for h, a in enumerate(a_list):
        asrc = asrc.at[h * f:(h + 1) * f, h].set(a[:f, 0])
        adst = adst.at[h * f:(h + 1) * f, h].set(a[f:, 0])
    return asrc, adst


def _factors(src, dst):
    md = jnp.max(dst, axis=0, keepdims=True)
    z = src + md
    e1 = jnp.exp(jnp.minimum(0.0, 0.8 * z))
    e2 = jnp.exp(-jnp.maximum(0.0, 0.8 * z))
    f1 = jnp.exp(dst - md)
    f2 = jnp.exp(0.2 * (dst - md))
    n_p = src.shape[0]
    aux = jnp.concatenate([e1, e2, jnp.zeros((n_p, 112), jnp.float32)], axis=1)
    fcol = jnp.concatenate([f1, f2], axis=1).T
    return aux, fcol


def _whx_of(wh, valid, nh, f):
    n_p = wh.shape[0]
    whx = jnp.zeros((n_p, nh, 128), jnp.float32)
    whx = whx.at[:, :, :f].set(wh.reshape(n_p, nh, f))
    whx = whx.at[:, :, f].set(valid[:, None])
    return whx.reshape(n_p, nh * 128).astype(jnp.bfloat16)


def kernel(x, adj, W0, a0, W1, a1, W2, a2, W3, a3, Wout, aout):
    n = x.shape[0]
    f = W0.shape[1]
    n_p = ((n + _BJ - 1) // _BJ) * _BJ
    valid = jnp.pad(jnp.ones((n,), jnp.float32), (0, n_p - n))

    wcat = jnp.concatenate([W0, W1, W2, W3], axis=1)
    asrc1, adst1 = _block_diag_a([a0, a1, a2, a3], f)
    x_p = jnp.pad(x, ((0, n_p - n), (0, 0)))
    wh1, src1, dst1, swh1 = _proj(x_p, wcat, asrc1, adst1)
    aux1, fcol1 = _factors(src1, dst1)
    h1 = _flash(aux1, fcol1, _whx_of(wh1, valid, 4, f), swh1, adj, nh=4, f=f)

    f2 = Wout.shape[1]
    asrc2, adst2 = _block_diag_a([aout], f2)
    h1_p = jnp.pad(h1, ((0, n_p - n), (0, 0)))
    wh2, src2, dst2, swh2 = _proj(h1_p, Wout, asrc2, adst2)
    aux2, fcol2 = _factors(src2, dst2)
    return _flash(aux2, fcol2, _whx_of(wh2, valid, 1, f2), swh2, adj, nh=1, f=f2)
